# baseline (speedup 1.0000x reference)
import functools

import jax
import jax.numpy as jnp
from jax import lax
from jax.experimental import pallas as pl
from jax.experimental.pallas import tpu as pltpu
from jax.experimental.pallas import tpu_sc as plsc

_THRES = 0.5
_L = 4096
_BR = 256
_NB = _L // _BR
_NW = 32
_IDX_TOTAL = 3 * _L
_PER_W = _IDX_TOTAL // _NW
_CHUNK = 128


def _rowstat_body(x_ref, base_ref, idx1_ref, idx2_ref, idxd_ref):
    pi = pl.program_id(0)
    x = x_ref[...]
    col = jax.lax.broadcasted_iota(jnp.int32, (_BR, _L), 1)
    g = pi * _BR + jax.lax.broadcasted_iota(jnp.int32, (_BR, 1), 0)
    masked = jnp.where(col == g, -jnp.inf, x)
    vmax = jnp.max(masked, axis=1, keepdims=True)
    cand = jnp.where(masked == vmax, col, _L)
    inds = jnp.min(cand, axis=1, keepdims=True)
    m = vmax > _THRES
    diag = g * (_L + 1)
    idx1 = jnp.where(m, g * _L + inds, diag)
    idx2 = jnp.where(m, inds * _L + g, diag)
    base_ref[...] = jnp.zeros((_BR, _L), jnp.float32)
    idx1_ref[...] = idx1[None]
    idx2_ref[...] = idx2[None]
    idxd_ref[...] = diag[None]


_sc_mesh = plsc.VectorSubcoreMesh(core_axis_name="c", subcore_axis_name="s")


@functools.partial(
    pl.kernel,
    out_type=(),
    mesh=_sc_mesh,
    scratch_types=[
        pltpu.VMEM((_CHUNK,), jnp.int32),
        pltpu.VMEM((_CHUNK,), jnp.float32),
        pltpu.SemaphoreType.DMA,
    ],
)
def _sc_scatter(idx_hbm, out_ref, idx_v, ones_v, sem):
    wid = lax.axis_index("s") * 2 + lax.axis_index("c")
    for t in range(_CHUNK // 16):
        ones_v[pl.ds(t * 16, 16)] = jnp.full((16,), 1.0, jnp.float32)
    base = wid * _PER_W
    for k in range(_PER_W // _CHUNK):
        pltpu.sync_copy(idx_hbm.at[pl.ds(base + k * _CHUNK, _CHUNK)], idx_v)
        pltpu.async_copy(ones_v, out_ref.at[idx_v], sem).wait()


def kernel(input):
    x = input.reshape(_L, _L)

    idx_spec = pl.BlockSpec((1, _BR, 1), lambda i: (i, 0, 0))
    idx_shape = jax.ShapeDtypeStruct((_NB, _BR, 1), jnp.int32)
    base, idx1, idx2, idxd = pl.pallas_call(
        _rowstat_body,
        grid=(_NB,),
        in_specs=[pl.BlockSpec((_BR, _L), lambda i: (i, 0))],
        out_specs=[
            pl.BlockSpec((_BR, _L), lambda i: (i, 0)),
            idx_spec,
            idx_spec,
            idx_spec,
        ],
        out_shape=[
            jax.ShapeDtypeStruct((_L, _L), jnp.float32),
            idx_shape,
            idx_shape,
            idx_shape,
        ],
    )(x)

    idx_all = jnp.concatenate(
        [idx1.reshape(_L), idx2.reshape(_L), idxd.reshape(_L)]
    )

    out_ref = jax.new_ref(base.reshape(_L * _L))
    _sc_scatter(idx_all, out_ref)
    return out_ref[...].reshape(input.shape)
